# baseline (speedup 1.0000x reference)
import functools

import jax
import jax.numpy as jnp
from jax import lax
from jax.experimental import pallas as pl
from jax.experimental.pallas import tpu as pltpu
from jax.experimental.pallas import tpu_sc as plsc

_N_FIELDS = 26
_VOCAB = 100000
_N_ROWS = _VOCAB + 1
_EMBED = 16
_BATCH = 16384

_NC = 2
_NS = 16
_NW = _NC * _NS

_ROWS_PER_W = _BATCH // _NW
_CHUNK = 128
_N_CHUNKS = _ROWS_PER_W // _CHUNK
_CELEMS = _CHUNK * _N_FIELDS
_L = 16


def _body(z_hbm, tab_hbm, out_hbm, z_v, idx_v, rows_v, sem):
    wid = lax.axis_index("s") * _NC + lax.axis_index("c")

    def do_chunk(g, _):
        base = (wid * _N_CHUNKS + g) * _CELEMS

        pltpu.sync_copy(z_hbm.at[pl.ds(base, _CELEMS)], z_v)

        def idx_step(j, _):
            for u in range(8):
                off = j * 128 + u * _L
                zv = z_v[pl.ds(off, _L)]
                nan = zv != zv
                zi = jnp.where(nan, jnp.float32(0), zv).astype(jnp.int32)
                row = jnp.where(nan, jnp.int32(_VOCAB), zi)
                pos = off + lax.iota(jnp.int32, _L)
                fld = lax.rem(pos, _N_FIELDS)
                idx_v[pl.ds(off, _L)] = row + fld * _N_ROWS
            return ()

        lax.fori_loop(0, _CELEMS // 128, idx_step, (), unroll=False)

        pltpu.async_copy(tab_hbm.at[idx_v], rows_v, sem).wait()

        pltpu.sync_copy(rows_v, out_hbm.at[pl.ds(base, _CELEMS)])
        return ()

    lax.fori_loop(0, _N_CHUNKS, do_chunk, (), unroll=True)


@jax.jit
def _embed_lookup(z_flat, tab_flat):
    mesh = plsc.VectorSubcoreMesh(core_axis_name="c", subcore_axis_name="s")
    kfn = pl.kernel(
        _body,
        out_type=jax.ShapeDtypeStruct((_BATCH * _N_FIELDS, _EMBED), jnp.float32),
        mesh=mesh,
        scratch_types=[
            pltpu.VMEM((_CELEMS,), jnp.float32),
            pltpu.VMEM((_CELEMS,), jnp.int32),
            pltpu.VMEM((_CELEMS, _EMBED), jnp.float32),
            pltpu.SemaphoreType.DMA,
        ],
        compiler_params=pltpu.CompilerParams(use_tc_tiling_on_sc=False),
    )
    return kfn(z_flat, tab_flat)


def kernel(Z_vec, tables):
    z_flat = Z_vec.reshape(_BATCH * _N_FIELDS)
    tab_flat = tables.reshape(_N_FIELDS * _N_ROWS, _EMBED)
    out = _embed_lookup(z_flat, tab_flat)
    return out.reshape(_BATCH, _N_FIELDS * _EMBED)


# speedup vs baseline: 1.9802x; 1.9802x over previous
import jax
import jax.numpy as jnp
from jax import lax
from jax.experimental import pallas as pl
from jax.experimental.pallas import tpu as pltpu
from jax.experimental.pallas import tpu_sc as plsc

_N_FIELDS = 26
_VOCAB = 100000
_N_ROWS = _VOCAB + 1
_EMBED = 16
_BATCH = 16384

_NW = 32
_ROWS_PER_W = _BATCH // _NW
_CHUNK = 128
_N_CHUNKS = _ROWS_PER_W // _CHUNK
_L = 16


def _idx_body(z_ref, idx_ref):
    z = z_ref[...]
    idx = jnp.where(jnp.isnan(z), jnp.int32(_VOCAB),
                    jnp.where(jnp.isnan(z), jnp.float32(0), z).astype(jnp.int32))
    idx_ref[...] = idx.T


def _gather_body(idx_hbm, tab_hbm, out_hbm, idx_v, rows_v, sem, gsem):
    wid = lax.axis_index("s") * 2 + lax.axis_index("c")

    def do_chunk(g, _):
        b0 = (wid * _N_CHUNKS + g) * _CHUNK

        pltpu.sync_copy(idx_hbm.at[:, pl.ds(b0, _CHUNK)], idx_v)

        gathers = [
            pltpu.async_copy(tab_hbm.at[f].at[idx_v.at[f]], rows_v.at[f], gsem)
            for f in range(_N_FIELDS)
        ]
        for c in gathers:
            c.wait()

        outs = [
            pltpu.async_copy(
                rows_v.at[f],
                out_hbm.at[pl.ds(b0, _CHUNK), pl.ds(f * _EMBED, _EMBED)],
                sem,
            )
            for f in range(_N_FIELDS)
        ]
        for c in outs:
            c.wait()
        return ()

    lax.fori_loop(0, _N_CHUNKS, do_chunk, (), unroll=False)


@jax.jit
def _embed_lookup(Z_vec, tables):
    idxT = pl.pallas_call(
        _idx_body,
        out_shape=jax.ShapeDtypeStruct((_N_FIELDS, _BATCH), jnp.int32),
    )(Z_vec)

    mesh = plsc.VectorSubcoreMesh(core_axis_name="c", subcore_axis_name="s")
    kfn = pl.kernel(
        _gather_body,
        out_type=jax.ShapeDtypeStruct((_BATCH, _N_FIELDS * _EMBED), jnp.float32),
        mesh=mesh,
        scratch_types=[
            pltpu.VMEM((_N_FIELDS, _CHUNK), jnp.int32),
            pltpu.VMEM((_N_FIELDS, _CHUNK, _EMBED), jnp.float32),
            pltpu.SemaphoreType.DMA,
            pltpu.SemaphoreType.DMA,
        ],
        compiler_params=pltpu.CompilerParams(use_tc_tiling_on_sc=False),
    )
    return kfn(idxT, tables)


def kernel(Z_vec, tables):
    return _embed_lookup(Z_vec, tables)


# speedup vs baseline: 23.5263x; 11.8806x over previous
import jax
import jax.numpy as jnp
from jax import lax
from jax.experimental import pallas as pl
from jax.experimental.pallas import tpu as pltpu
from jax.experimental.pallas import tpu_sc as plsc

_N_FIELDS = 26
_VOCAB = 100000
_N_ROWS = _VOCAB + 1
_EMBED = 16
_BATCH = 16384

_NW = 32
_NU = _N_FIELDS * _EMBED
_UPT = _NU // _NW
_BC = 2048
_L = 16


def _idx_body(z_ref, i_ref):
    f = pl.program_id(0)
    z = z_ref[pl.ds(f, 1)][0]
    nan = jnp.isnan(z)
    i_ref[...] = jnp.where(nan, jnp.int32(_VOCAB),
                           jnp.where(nan, jnp.float32(0), z).astype(jnp.int32))


def _lookup_body(idx_hbm, tab_hbm, out_hbm, plane_v, idxc_v, obuf_v, sem):
    wid = lax.axis_index("s") * 2 + lax.axis_index("c")

    def do_unit(k, _):
        u = wid * _UPT + k
        f = u // _EMBED
        e = u % _EMBED

        pltpu.sync_copy(tab_hbm.at[f, e], plane_v)

        def do_chunk(c, _):
            b0 = c * _BC
            pltpu.sync_copy(idx_hbm.at[pl.ds(f * _BATCH + b0, _BC)], idxc_v)

            def gat(j, _):
                v = idxc_v[pl.ds(j * _L, _L)]
                obuf_v[pl.ds(j * _L, _L)] = plsc.load_gather(plane_v, [v])
                return ()

            lax.fori_loop(0, _BC // _L, gat, (), unroll=4)

            pltpu.sync_copy(obuf_v, out_hbm.at[u, pl.ds(b0, _BC)])
            return ()

        lax.fori_loop(0, _BATCH // _BC, do_chunk, (), unroll=False)
        return ()

    lax.fori_loop(0, _UPT, do_unit, (), unroll=False)


@jax.jit
def _embed_lookup(Z_vec, tables):
    tab_t = jnp.transpose(tables, (0, 2, 1))
    z_t = jnp.transpose(Z_vec)

    idx1d = pl.pallas_call(
        _idx_body,
        grid=(_N_FIELDS,),
        in_specs=[pl.BlockSpec((_N_FIELDS, _BATCH), lambda f: (0, 0))],
        out_specs=pl.BlockSpec((_BATCH,), lambda f: (f,)),
        out_shape=jax.ShapeDtypeStruct((_N_FIELDS * _BATCH,), jnp.int32),
    )(z_t)

    mesh = plsc.VectorSubcoreMesh(core_axis_name="c", subcore_axis_name="s")
    kfn = pl.kernel(
        _lookup_body,
        out_type=jax.ShapeDtypeStruct((_NU, _BATCH), jnp.float32),
        mesh=mesh,
        scratch_types=[
            pltpu.VMEM((_N_ROWS,), jnp.float32),
            pltpu.VMEM((_BC,), jnp.int32),
            pltpu.VMEM((_BC,), jnp.float32),
            pltpu.SemaphoreType.DMA,
        ],
        compiler_params=pltpu.CompilerParams(
            use_tc_tiling_on_sc=True, needs_layout_passes=False),
    )
    out_t = kfn(idx1d, tab_t)
    return jnp.transpose(out_t)


def kernel(Z_vec, tables):
    return _embed_lookup(Z_vec, tables)


# speedup vs baseline: 26.3730x; 1.1210x over previous
import jax
import jax.numpy as jnp
from jax import lax
from jax.experimental import pallas as pl
from jax.experimental.pallas import tpu as pltpu
from jax.experimental.pallas import tpu_sc as plsc

_N_FIELDS = 26
_VOCAB = 100000
_N_ROWS = _VOCAB + 1
_EMBED = 16
_BATCH = 16384

_NW = 32
_NU = _N_FIELDS * _EMBED
_UPT = _NU // _NW
_BC = 4096
_L = 16


def _idx_body(z_ref, i_ref):
    f = pl.program_id(0)
    z = z_ref[pl.ds(f, 1)][0]
    nan = jnp.isnan(z)
    i_ref[...] = jnp.where(nan, jnp.int32(_VOCAB),
                           jnp.where(nan, jnp.float32(0), z).astype(jnp.int32))


def _lookup_body(idx_hbm, tab_hbm, out_hbm, plane_v, idxc_v, obuf_v, sem):
    wid = lax.axis_index("s") * 2 + lax.axis_index("c")

    def do_unit(k, _):
        u = wid * _UPT + k
        f = u // _EMBED
        e = u % _EMBED

        pltpu.sync_copy(tab_hbm.at[f, e], plane_v)

        def do_chunk(c, _):
            b0 = c * _BC
            pltpu.sync_copy(idx_hbm.at[pl.ds(f * _BATCH + b0, _BC)], idxc_v)

            def gat(j, _):
                v = idxc_v[pl.ds(j * _L, _L)]
                obuf_v[pl.ds(j * _L, _L)] = plsc.load_gather(plane_v, [v])
                return ()

            lax.fori_loop(0, _BC // _L, gat, (), unroll=8)

            pltpu.sync_copy(obuf_v, out_hbm.at[u, pl.ds(b0, _BC)])
            return ()

        lax.fori_loop(0, _BATCH // _BC, do_chunk, (), unroll=False)
        return ()

    lax.fori_loop(0, _UPT, do_unit, (), unroll=False)


@jax.jit
def _embed_lookup(Z_vec, tables):
    tab_t = jnp.transpose(tables, (0, 2, 1))
    z_t = jnp.transpose(Z_vec)

    idx1d = pl.pallas_call(
        _idx_body,
        grid=(_N_FIELDS,),
        in_specs=[pl.BlockSpec((_N_FIELDS, _BATCH), lambda f: (0, 0))],
        out_specs=pl.BlockSpec((_BATCH,), lambda f: (f,)),
        out_shape=jax.ShapeDtypeStruct((_N_FIELDS * _BATCH,), jnp.int32),
    )(z_t)

    mesh = plsc.VectorSubcoreMesh(core_axis_name="c", subcore_axis_name="s")
    kfn = pl.kernel(
        _lookup_body,
        out_type=jax.ShapeDtypeStruct((_NU, _BATCH), jnp.float32),
        mesh=mesh,
        scratch_types=[
            pltpu.VMEM((_N_ROWS,), jnp.float32),
            pltpu.VMEM((_BC,), jnp.int32),
            pltpu.VMEM((_BC,), jnp.float32),
            pltpu.SemaphoreType.DMA,
        ],
        compiler_params=pltpu.CompilerParams(
            use_tc_tiling_on_sc=True, needs_layout_passes=False),
    )
    out_t = kfn(idx1d, tab_t)
    return jnp.transpose(out_t)


def kernel(Z_vec, tables):
    return _embed_lookup(Z_vec, tables)


# speedup vs baseline: 30.1952x; 1.1449x over previous
import jax
import jax.numpy as jnp
from jax import lax
from jax.experimental import pallas as pl
from jax.experimental.pallas import tpu as pltpu
from jax.experimental.pallas import tpu_sc as plsc

_N_FIELDS = 26
_VOCAB = 100000
_N_ROWS = _VOCAB + 1
_EMBED = 16
_BATCH = 16384

_NW = 32
_NU = _N_FIELDS * _EMBED
_UPT = _NU // _NW
_BC = 4096
_L = 16


def _idx_body(z_ref, i_ref):
    f = pl.program_id(0)
    z = z_ref[pl.ds(f, 1)][0]
    nan = jnp.isnan(z)
    i_ref[...] = jnp.where(nan, jnp.int32(_VOCAB),
                           jnp.where(nan, jnp.float32(0), z).astype(jnp.int32))


_NCH = _BATCH // _BC


def _lookup_body(idx_hbm, tab_hbm, out_hbm, plane_v, idxc_v, obuf_v,
                 psem, isem, osem):
    wid = lax.axis_index("s") * 2 + lax.axis_index("c")

    def ufe(k):
        u = wid * _UPT + k
        return u, u // _EMBED, u % _EMBED

    _, f0, e0 = ufe(0)
    plane_cp = pltpu.async_copy(tab_hbm.at[f0, e0], plane_v, psem)
    idx_cp = pltpu.async_copy(idx_hbm.at[pl.ds(f0 * _BATCH, _BC)],
                              idxc_v.at[0], isem)
    out_cps = []

    for k in range(_UPT):
        u, f, e = ufe(k)
        plane_cp.wait()
        for c in range(_NCH):
            buf = c % 2
            idx_cp.wait()
            if c + 1 < _NCH:
                idx_cp = pltpu.async_copy(
                    idx_hbm.at[pl.ds(f * _BATCH + (c + 1) * _BC, _BC)],
                    idxc_v.at[1 - buf], isem)
            elif k + 1 < _UPT:
                _, f2, _ = ufe(k + 1)
                idx_cp = pltpu.async_copy(
                    idx_hbm.at[pl.ds(f2 * _BATCH, _BC)],
                    idxc_v.at[1 - buf], isem)
            if len(out_cps) >= 2:
                out_cps[-2].wait()

            def gat(j, _):
                v = idxc_v[buf, pl.ds(j * _L, _L)]
                obuf_v[buf, pl.ds(j * _L, _L)] = plsc.load_gather(plane_v, [v])
                return ()

            lax.fori_loop(0, _BC // _L, gat, (), unroll=8)

            if c == _NCH - 1 and k + 1 < _UPT:
                _, f2, e2 = ufe(k + 1)
                plane_cp = pltpu.async_copy(tab_hbm.at[f2, e2], plane_v, psem)
            out_cps.append(pltpu.async_copy(
                obuf_v.at[buf], out_hbm.at[u, pl.ds(c * _BC, _BC)], osem))

    for cp in out_cps[-2:]:
        cp.wait()


@jax.jit
def _embed_lookup(Z_vec, tables):
    tab_t = jnp.transpose(tables, (0, 2, 1))
    z_t = jnp.transpose(Z_vec)

    idx1d = pl.pallas_call(
        _idx_body,
        grid=(_N_FIELDS,),
        in_specs=[pl.BlockSpec((_N_FIELDS, _BATCH), lambda f: (0, 0))],
        out_specs=pl.BlockSpec((_BATCH,), lambda f: (f,)),
        out_shape=jax.ShapeDtypeStruct((_N_FIELDS * _BATCH,), jnp.int32),
    )(z_t)

    mesh = plsc.VectorSubcoreMesh(core_axis_name="c", subcore_axis_name="s")
    kfn = pl.kernel(
        _lookup_body,
        out_type=jax.ShapeDtypeStruct((_NU, _BATCH), jnp.float32),
        mesh=mesh,
        scratch_types=[
            pltpu.VMEM((_N_ROWS,), jnp.float32),
            pltpu.VMEM((2, _BC), jnp.int32),
            pltpu.VMEM((2, _BC), jnp.float32),
            pltpu.SemaphoreType.DMA,
            pltpu.SemaphoreType.DMA,
            pltpu.SemaphoreType.DMA,
        ],
        compiler_params=pltpu.CompilerParams(
            use_tc_tiling_on_sc=True, needs_layout_passes=False),
    )
    out_t = kfn(idx1d, tab_t)
    return jnp.transpose(out_t)


def kernel(Z_vec, tables):
    return _embed_lookup(Z_vec, tables)


# speedup vs baseline: 31.1495x; 1.0316x over previous
import jax
import jax.numpy as jnp
from jax import lax
from jax.experimental import pallas as pl
from jax.experimental.pallas import tpu as pltpu
from jax.experimental.pallas import tpu_sc as plsc

_N_FIELDS = 26
_VOCAB = 100000
_N_ROWS = _VOCAB + 1
_EMBED = 16
_BATCH = 16384

_NW = 32
_NU = _N_FIELDS * _EMBED
_UPT = _NU // _NW
_BC = 4096
_L = 16


def _idx_body(z_ref, i_ref):
    f = pl.program_id(0)
    z = z_ref[pl.ds(f, 1)][0]
    nan = jnp.isnan(z)
    i_ref[...] = jnp.where(nan, jnp.int32(_VOCAB),
                           jnp.where(nan, jnp.float32(0), z).astype(jnp.int32))


_NCH = _BATCH // _BC


def _lookup_body(idx_hbm, tab_hbm, out_hbm, plane_v, idxf_v, obuf_v,
                 psem, isem, osem):
    wid = lax.axis_index("s") * 2 + lax.axis_index("c")

    def ufe(k):
        u = wid * _UPT + k
        return u, u // _EMBED, u % _EMBED

    _, f0, e0 = ufe(0)
    plane_cp = pltpu.async_copy(tab_hbm.at[f0, e0], plane_v, psem)
    pltpu.sync_copy(idx_hbm.at[pl.ds(f0 * _BATCH, _BATCH)], idxf_v)
    out_cps = []

    for k in range(_UPT):
        u, f, e = ufe(k)
        if k > 0:
            @pl.when(e == 0)
            def _():
                pltpu.sync_copy(idx_hbm.at[pl.ds(f * _BATCH, _BATCH)], idxf_v)
        plane_cp.wait()
        for c in range(_NCH):
            buf = c % 2
            if len(out_cps) >= 2:
                out_cps[-2].wait()
            base = c * _BC

            def gat(j, _):
                v = idxf_v[pl.ds(base + j * _L, _L)]
                obuf_v[buf, pl.ds(j * _L, _L)] = plsc.load_gather(plane_v, [v])
                return ()

            lax.fori_loop(0, _BC // _L, gat, (), unroll=8)

            if c == _NCH - 1 and k + 1 < _UPT:
                _, f2, e2 = ufe(k + 1)
                plane_cp = pltpu.async_copy(tab_hbm.at[f2, e2], plane_v, psem)
            out_cps.append(pltpu.async_copy(
                obuf_v.at[buf], out_hbm.at[u, pl.ds(base, _BC)], osem))

    for cp in out_cps[-2:]:
        cp.wait()


@jax.jit
def _embed_lookup(Z_vec, tables):
    tab_t = jnp.transpose(tables, (0, 2, 1))
    z_t = jnp.transpose(Z_vec)

    idx1d = pl.pallas_call(
        _idx_body,
        grid=(_N_FIELDS,),
        in_specs=[pl.BlockSpec((_N_FIELDS, _BATCH), lambda f: (0, 0))],
        out_specs=pl.BlockSpec((_BATCH,), lambda f: (f,)),
        out_shape=jax.ShapeDtypeStruct((_N_FIELDS * _BATCH,), jnp.int32),
    )(z_t)

    mesh = plsc.VectorSubcoreMesh(core_axis_name="c", subcore_axis_name="s")
    kfn = pl.kernel(
        _lookup_body,
        out_type=jax.ShapeDtypeStruct((_NU, _BATCH), jnp.float32),
        mesh=mesh,
        scratch_types=[
            pltpu.VMEM((_N_ROWS,), jnp.float32),
            pltpu.VMEM((_BATCH,), jnp.int32),
            pltpu.VMEM((2, _BC), jnp.float32),
            pltpu.SemaphoreType.DMA,
            pltpu.SemaphoreType.DMA,
            pltpu.SemaphoreType.DMA,
        ],
        compiler_params=pltpu.CompilerParams(
            use_tc_tiling_on_sc=True, needs_layout_passes=False),
    )
    out_t = kfn(idx1d, tab_t)
    return jnp.transpose(out_t)


def kernel(Z_vec, tables):
    return _embed_lookup(Z_vec, tables)
